# baseline (speedup 1.0000x reference)
import functools

import numpy as np
import jax
import jax.numpy as jnp
from jax import lax
from jax.experimental import pallas as pl
from jax.experimental.pallas import tpu as pltpu
from jax.experimental.pallas import tpu_sc as plsc

VOCAB_SIZE = 100000
EMB_DIM = 1024
SEQ_LEN = 2048
NUM_SEQ = 4
SCALE = 32.0

_NUM_CORES = 2
_NUM_SUBCORES = 16
_LANES = 16
_NW = _NUM_CORES * _NUM_SUBCORES
_B = NUM_SEQ * SEQ_LEN
_POS_PER_W = SEQ_LEN // _NW
_ROWS_PER_W = _POS_PER_W * NUM_SEQ
_CHUNK = 16
_N_SUB = _POS_PER_W // _CHUNK
_N_CHUNKS = _N_SUB * NUM_SEQ


def _make_pos_encoding(length, depth):
    half = depth / 2
    positions = np.reshape(np.arange(length), [-1, 1])
    depths = np.expand_dims(np.arange(half), axis=0) / half
    angle_rads = positions * (1 / 10000 ** depths)
    return np.concatenate(
        [np.sin(angle_rads), np.cos(angle_rads)], axis=-1
    ).astype(np.float32)


_PE = _make_pos_encoding(SEQ_LEN, EMB_DIM)

_mesh = plsc.VectorSubcoreMesh(core_axis_name="c", subcore_axis_name="s")


@functools.partial(
    pl.kernel,
    mesh=_mesh,
    out_type=jax.ShapeDtypeStruct((_B, EMB_DIM), jnp.float32),
    scratch_types=[
        pltpu.VMEM((_ROWS_PER_W,), jnp.int32),
        pltpu.VMEM((_CHUNK, EMB_DIM), jnp.float32),
        pltpu.VMEM((_CHUNK, EMB_DIM), jnp.float32),
        pltpu.VMEM((_CHUNK, EMB_DIM), jnp.float32),
        pltpu.VMEM((_CHUNK, EMB_DIM), jnp.float32),
        pltpu.VMEM((_CHUNK, EMB_DIM), jnp.float32),
        pltpu.SemaphoreType.DMA,
        pltpu.SemaphoreType.DMA,
        pltpu.SemaphoreType.DMA,
        pltpu.SemaphoreType.DMA,
    ],
)
def _sc_embed(table_hbm, idx_hbm, pe_hbm, out_hbm,
              idx_v, in0, in1, ot0, ot1, pe_v,
              g0, g1, s0, s1):
    ins, outs = (in0, in1), (ot0, ot1)
    gsem, ssem = (g0, g1), (s0, s1)

    wid = lax.axis_index("s") * _NUM_CORES + lax.axis_index("c")
    pos0 = wid * _POS_PER_W

    for b in range(NUM_SEQ):
        pltpu.sync_copy(
            idx_hbm.at[pl.ds(b * SEQ_LEN + pos0, _POS_PER_W)],
            idx_v.at[pl.ds(b * _POS_PER_W, _POS_PER_W)])

    def flat_row(psub, b):
        return b * SEQ_LEN + pos0 + psub * _CHUNK

    def issue_gather(psub, b, k):
        idx_vec = idx_v[pl.ds(b * _POS_PER_W + psub * _CHUNK, _CHUNK)]
        pltpu.async_copy(table_hbm.at[idx_vec], ins[k], gsem[k])

    pltpu.sync_copy(pe_hbm.at[pl.ds(pos0, _CHUNK)], pe_v)
    issue_gather(0, 0, 0)
    issue_gather(0, 1, 1)

    def sub_body(psub, carry):
        for b in range(NUM_SEQ):
            k = b % 2

            if b == 0:
                @pl.when(psub >= 1)
                def _load_pe():
                    pltpu.sync_copy(
                        pe_hbm.at[pl.ds(pos0 + psub * _CHUNK, _CHUNK)],
                        pe_v)

            pltpu.make_async_copy(
                table_hbm.at[idx_v[pl.ds(0, _CHUNK)]], ins[k],
                gsem[k]).wait()

            if b >= 2:
                pltpu.make_async_copy(
                    outs[k], out_hbm.at[pl.ds(0, _CHUNK)], ssem[k]).wait()
            else:
                @pl.when(psub >= 1)
                def _wait_store():
                    pltpu.make_async_copy(
                        outs[k], out_hbm.at[pl.ds(0, _CHUNK)],
                        ssem[k]).wait()

            def row_body(r, rc):
                for j in range(EMB_DIM // _LANES):
                    sl = pl.ds(j * _LANES, _LANES)
                    outs[k][r, sl] = ins[k][r, sl] * SCALE + pe_v[r, sl]
                return rc

            lax.fori_loop(0, _CHUNK, row_body, 0)

            pltpu.async_copy(
                outs[k], out_hbm.at[pl.ds(flat_row(psub, b), _CHUNK)],
                ssem[k])

            if b < 2:
                issue_gather(psub, b + 2, k)
            else:
                @pl.when(psub < _N_SUB - 1)
                def _issue_next():
                    issue_gather(psub + 1, b - 2, k)
        return carry

    lax.fori_loop(0, _N_SUB, sub_body, 0)

    for k in range(2):
        pltpu.make_async_copy(
            outs[k], out_hbm.at[pl.ds(0, _CHUNK)], ssem[k]).wait()


def kernel(x, table):
    idx = x.reshape(-1).astype(jnp.int32)
    pe = jnp.asarray(_PE)
    out = _sc_embed(table, idx, pe)
    return out.reshape(NUM_SEQ, SEQ_LEN, EMB_DIM)


# speedup vs baseline: 1.0400x; 1.0400x over previous
import functools

import numpy as np
import jax
import jax.numpy as jnp
from jax import lax
from jax.experimental import pallas as pl
from jax.experimental.pallas import tpu as pltpu
from jax.experimental.pallas import tpu_sc as plsc

VOCAB_SIZE = 100000
EMB_DIM = 1024
SEQ_LEN = 2048
NUM_SEQ = 4
SCALE = 32.0

_NUM_CORES = 2
_NUM_SUBCORES = 16
_LANES = 16
_NW = _NUM_CORES * _NUM_SUBCORES
_B = NUM_SEQ * SEQ_LEN
_ROWS_PER_W = _B // _NW
_CHUNK = 16
_N_CHUNKS = _ROWS_PER_W // _CHUNK


def _make_pos_encoding(length, depth):
    half = depth / 2
    positions = np.reshape(np.arange(length), [-1, 1])
    depths = np.expand_dims(np.arange(half), axis=0) / half
    angle_rads = positions * (1 / 10000 ** depths)
    return np.concatenate(
        [np.sin(angle_rads), np.cos(angle_rads)], axis=-1
    ).astype(np.float32)


_PE = _make_pos_encoding(SEQ_LEN, EMB_DIM)

_mesh = plsc.VectorSubcoreMesh(core_axis_name="c", subcore_axis_name="s")


@functools.partial(
    pl.kernel,
    mesh=_mesh,
    out_type=jax.ShapeDtypeStruct((_B, EMB_DIM), jnp.float32),
    scratch_types=[
        pltpu.VMEM((_ROWS_PER_W,), jnp.int32),
        pltpu.VMEM((_CHUNK, EMB_DIM), jnp.float32),
        pltpu.VMEM((_CHUNK, EMB_DIM), jnp.float32),
        pltpu.VMEM((_CHUNK, EMB_DIM), jnp.float32),
        pltpu.VMEM((_CHUNK, EMB_DIM), jnp.float32),
        pltpu.VMEM((_CHUNK, EMB_DIM), jnp.float32),
        pltpu.VMEM((_CHUNK, EMB_DIM), jnp.float32),
        pltpu.SemaphoreType.DMA,
        pltpu.SemaphoreType.DMA,
        pltpu.SemaphoreType.DMA,
        pltpu.SemaphoreType.DMA,
        pltpu.SemaphoreType.DMA,
        pltpu.SemaphoreType.DMA,
        pltpu.SemaphoreType.DMA,
        pltpu.SemaphoreType.DMA,
        pltpu.SemaphoreType.DMA,
        pltpu.SemaphoreType.DMA,
    ],
)
def _sc_embed(table_hbm, idx_hbm, pe_hbm, out_hbm,
              idx_v, in0, in1, ub0, ub1, ub2, ub3,
              g0, g1, p0, p1, p2, p3, s0, s1, s2, s3):
    ins = (in0, in1)
    ubs = (ub0, ub1, ub2, ub3)
    gsem = (g0, g1)
    psem = (p0, p1, p2, p3)
    ssem = (s0, s1, s2, s3)

    wid = lax.axis_index("s") * _NUM_CORES + lax.axis_index("c")
    base = wid * _ROWS_PER_W
    pe_base = lax.rem(base, SEQ_LEN)

    pltpu.sync_copy(idx_hbm.at[pl.ds(base, _ROWS_PER_W)], idx_v)

    def issue_gather(c, k):
        idx_vec = idx_v[pl.ds(c * _CHUNK, _CHUNK)]
        pltpu.async_copy(table_hbm.at[idx_vec], ins[k], gsem[k])

    def issue_pe(c, j):
        pltpu.async_copy(
            pe_hbm.at[pl.ds(pe_base + c * _CHUNK, _CHUNK)], ubs[j],
            psem[j])

    issue_pe(0, 0)
    issue_pe(1, 1)
    issue_gather(0, 0)
    issue_gather(1, 1)

    def loop_body(i, carry):
        for u in range(4):
            c = 4 * i + u
            k = u % 2
            j = u

            pltpu.make_async_copy(
                table_hbm.at[idx_v[pl.ds(0, _CHUNK)]], ins[k],
                gsem[k]).wait()
            pltpu.make_async_copy(
                pe_hbm.at[pl.ds(0, _CHUNK)], ubs[j], psem[j]).wait()

            def row_body(r, rc):
                for jj in range(EMB_DIM // _LANES):
                    sl = pl.ds(jj * _LANES, _LANES)
                    plsc.addupdate(
                        ubs[j].at[r, sl], ins[k][r, sl] * SCALE)
                return rc

            lax.fori_loop(0, _CHUNK, row_body, 0)

            pltpu.async_copy(
                ubs[j], out_hbm.at[pl.ds(base + c * _CHUNK, _CHUNK)],
                ssem[j])

            jn = (u + 2) % 4
            if u < 2:
                @pl.when(i >= 1)
                def _wait_store():
                    pltpu.make_async_copy(
                        ubs[jn], out_hbm.at[pl.ds(0, _CHUNK)],
                        ssem[jn]).wait()

                issue_pe(c + 2, jn)
                issue_gather(c + 2, k)
            else:
                pltpu.make_async_copy(
                    ubs[jn], out_hbm.at[pl.ds(0, _CHUNK)],
                    ssem[jn]).wait()

                @pl.when(i < _N_CHUNKS // 4 - 1)
                def _issue_next():
                    issue_pe(c + 2, jn)
                    issue_gather(c + 2, k)
        return carry

    lax.fori_loop(0, _N_CHUNKS // 4, loop_body, 0)

    for j in (2, 3):
        pltpu.make_async_copy(
            ubs[j], out_hbm.at[pl.ds(0, _CHUNK)], ssem[j]).wait()


def kernel(x, table):
    idx = x.reshape(-1).astype(jnp.int32)
    pe = jnp.asarray(_PE)
    out = _sc_embed(table, idx, pe)
    return out.reshape(NUM_SEQ, SEQ_LEN, EMB_DIM)


# speedup vs baseline: 1.0635x; 1.0226x over previous
import functools

import numpy as np
import jax
import jax.numpy as jnp
from jax import lax
from jax.experimental import pallas as pl
from jax.experimental.pallas import tpu as pltpu
from jax.experimental.pallas import tpu_sc as plsc

VOCAB_SIZE = 100000
EMB_DIM = 1024
SEQ_LEN = 2048
NUM_SEQ = 4
SCALE = 32.0

_NUM_CORES = 2
_NUM_SUBCORES = 16
_LANES = 16
_NW = _NUM_CORES * _NUM_SUBCORES
_B = NUM_SEQ * SEQ_LEN
_POS_PER_W = SEQ_LEN // _NW
_ROWS_PER_W = _POS_PER_W * NUM_SEQ
_CHUNK = 16
_N_SUB = _POS_PER_W // _CHUNK
_N_CHUNKS = _N_SUB * NUM_SEQ


def _make_pos_encoding(length, depth):
    half = depth / 2
    positions = np.reshape(np.arange(length), [-1, 1])
    depths = np.expand_dims(np.arange(half), axis=0) / half
    angle_rads = positions * (1 / 10000 ** depths)
    return np.concatenate(
        [np.sin(angle_rads), np.cos(angle_rads)], axis=-1
    ).astype(np.float32)


_PE = _make_pos_encoding(SEQ_LEN, EMB_DIM)

_mesh = plsc.VectorSubcoreMesh(core_axis_name="c", subcore_axis_name="s")


@functools.partial(
    pl.kernel,
    mesh=_mesh,
    out_type=jax.ShapeDtypeStruct((_B, EMB_DIM), jnp.float32),
    scratch_types=[
        pltpu.VMEM((_ROWS_PER_W,), jnp.int32),
        pltpu.VMEM((_CHUNK, EMB_DIM), jnp.float32),
        pltpu.VMEM((_CHUNK, EMB_DIM), jnp.float32),
        pltpu.VMEM((_CHUNK, EMB_DIM), jnp.float32),
        pltpu.VMEM((_CHUNK, EMB_DIM), jnp.float32),
        pltpu.VMEM((_CHUNK, EMB_DIM), jnp.float32),
        pltpu.VMEM((_CHUNK, EMB_DIM), jnp.float32),
        pltpu.SemaphoreType.DMA,
        pltpu.SemaphoreType.DMA,
        pltpu.SemaphoreType.DMA,
        pltpu.SemaphoreType.DMA,
        pltpu.SemaphoreType.DMA,
        pltpu.SemaphoreType.DMA,
    ],
)
def _sc_embed(table_hbm, idx_hbm, pe_hbm, out_hbm,
              idx_v, in0, in1, ot0, ot1, pe0, pe1,
              g0, g1, s0, s1, p0, p1):
    ins, outs, pes = (in0, in1), (ot0, ot1), (pe0, pe1)
    gsem, ssem, psem = (g0, g1), (s0, s1), (p0, p1)

    wid = lax.axis_index("s") * _NUM_CORES + lax.axis_index("c")
    pos0 = wid * _POS_PER_W

    for b in range(NUM_SEQ):
        pltpu.sync_copy(
            idx_hbm.at[pl.ds(b * SEQ_LEN + pos0, _POS_PER_W)],
            idx_v.at[pl.ds(b * _POS_PER_W, _POS_PER_W)])

    def issue_gather(psub, b, k):
        idx_vec = idx_v[pl.ds(b * _POS_PER_W + psub * _CHUNK, _CHUNK)]
        pltpu.async_copy(table_hbm.at[idx_vec], ins[k], gsem[k])

    def issue_pe(psub, m):
        pltpu.async_copy(
            pe_hbm.at[pl.ds(pos0 + psub * _CHUNK, _CHUNK)], pes[m],
            psem[m])

    issue_pe(0, 0)
    issue_pe(1, 1)
    issue_gather(0, 0, 0)
    issue_gather(0, 1, 1)

    def pair_body(ip, carry):
        for q in range(2):
            psub = 2 * ip + q
            m = q
            for b in range(NUM_SEQ):
                k = b % 2
                c_ge_2 = b >= 2 or q == 1

                if b == 0:
                    pltpu.make_async_copy(
                        pe_hbm.at[pl.ds(0, _CHUNK)], pes[m],
                        psem[m]).wait()

                pltpu.make_async_copy(
                    table_hbm.at[idx_v[pl.ds(0, _CHUNK)]], ins[k],
                    gsem[k]).wait()

                if c_ge_2:
                    pltpu.make_async_copy(
                        outs[k], out_hbm.at[pl.ds(0, _CHUNK)],
                        ssem[k]).wait()
                else:
                    @pl.when(ip >= 1)
                    def _wait_store():
                        pltpu.make_async_copy(
                            outs[k], out_hbm.at[pl.ds(0, _CHUNK)],
                            ssem[k]).wait()

                def row_body(r, rc):
                    for jj in range(EMB_DIM // _LANES):
                        sl = pl.ds(jj * _LANES, _LANES)
                        outs[k][r, sl] = (
                            ins[k][r, sl] * SCALE + pes[m][r, sl])
                    return rc

                lax.fori_loop(0, _CHUNK, row_body, 0)

                if b == NUM_SEQ - 1:
                    @pl.when(ip < 1)
                    def _next_pe():
                        issue_pe(psub + 2, m)

                pltpu.async_copy(
                    outs[k],
                    out_hbm.at[pl.ds(
                        b * SEQ_LEN + pos0 + psub * _CHUNK, _CHUNK)],
                    ssem[k])

                if b < 2:
                    issue_gather(psub, b + 2, k)
                elif q == 0:
                    issue_gather(psub + 1, b - 2, k)
                else:
                    @pl.when(ip < _N_SUB // 2 - 1)
                    def _issue_next():
                        issue_gather(psub + 1, b - 2, k)
        return carry

    lax.fori_loop(0, _N_SUB // 2, pair_body, 0)

    for k in range(2):
        pltpu.make_async_copy(
            outs[k], out_hbm.at[pl.ds(0, _CHUNK)], ssem[k]).wait()


def kernel(x, table):
    idx = x.reshape(-1).astype(jnp.int32)
    pe = jnp.asarray(_PE)
    out = _sc_embed(table, idx, pe)
    return out.reshape(NUM_SEQ, SEQ_LEN, EMB_DIM)
